# baseline (speedup 1.0000x reference)
import functools

import jax
import jax.numpy as jnp
from jax import lax
from jax.experimental import pallas as pl
from jax.experimental.pallas import tpu as pltpu
from jax.experimental.pallas import tpu_sc as plsc

_NC = 2
_NS = 16
_NT = _NC * _NS
_ECH = 7936


def _seg_sum_sc(table_t, packed, n_rows, d):
    e = packed.shape[0]
    unroll = 8
    assert e % _ECH == 0 and _ECH % (16 * unroll) == 0
    n_chunks = e // _ECH
    n_pass = d // _NT
    assert d % _NT == 0

    mesh = plsc.VectorSubcoreMesh(core_axis_name="c", subcore_axis_name="s")

    @functools.partial(
        pl.kernel,
        mesh=mesh,
        out_type=jax.ShapeDtypeStruct((d * n_rows,), jnp.float32),
        scratch_types=[
            pltpu.VMEM((n_rows,), jnp.float32),
            pltpu.VMEM((n_rows,), jnp.float32),
            pltpu.VMEM((2, _ECH), jnp.int32),
            pltpu.SemaphoreType.DMA,
        ],
        compiler_params=pltpu.CompilerParams(needs_layout_passes=False),
    )
    def seg_kernel(table_h, pk_h, out_h, tloc, aloc, pkb, sem):
        c = lax.axis_index("c")
        s = lax.axis_index("s")
        wid = s * _NC + c
        z16 = jnp.zeros((16,), jnp.float32)

        for p in range(n_pass):
            col = wid + _NT * p
            pltpu.sync_copy(table_h.at[pl.ds(col * n_rows, n_rows)], tloc)

            def zbody(i, carry):
                aloc[pl.ds(i * 16, 16)] = z16
                return carry

            lax.fori_loop(0, n_rows // 16, zbody, 0)

            pltpu.async_copy(pk_h.at[pl.ds(0, _ECH)], pkb.at[0], sem)

            def chunk_body(i, carry):
                par = i % 2
                nxt = (i + 1) % 2
                pltpu.make_async_copy(
                    pk_h.at[pl.ds(i * _ECH, _ECH)], pkb.at[par], sem).wait()

                @pl.when(i + 1 < n_chunks)
                def _prefetch():
                    pltpu.async_copy(
                        pk_h.at[pl.ds((i + 1) * _ECH, _ECH)], pkb.at[nxt],
                        sem)

                def ebody(g, carry2):
                    for u in range(unroll):
                        w = pkb[par, pl.ds((g * unroll + u) * 16, 16)]
                        s16 = jnp.bitwise_and(w, 0xFFFF)
                        d16 = lax.shift_right_logical(w, 16)
                        v = plsc.load_gather(tloc, [s16])
                        plsc.addupdate_scatter(aloc, [d16], v)
                    return carry2

                lax.fori_loop(0, _ECH // (16 * unroll), ebody, 0)
                return carry

            lax.fori_loop(0, n_chunks, chunk_body, 0)
            pltpu.sync_copy(aloc, out_h.at[pl.ds(col * n_rows, n_rows)])

    return seg_kernel(table_t, packed)


def _deg_sc(dst, n_rows):
    e = dst.shape[0]
    ept = e // _NT
    assert e % _NT == 0 and ept % 16 == 0

    mesh = plsc.VectorSubcoreMesh(core_axis_name="c", subcore_axis_name="s")

    @functools.partial(
        pl.kernel,
        mesh=mesh,
        out_type=jax.ShapeDtypeStruct((_NT * n_rows,), jnp.float32),
        scratch_types=[
            pltpu.VMEM((n_rows,), jnp.float32),
            pltpu.VMEM((ept,), jnp.int32),
            pltpu.SemaphoreType.DMA,
        ],
        compiler_params=pltpu.CompilerParams(needs_layout_passes=False),
    )
    def deg_kernel(dst_h, out_h, degloc, dstb, sem):
        c = lax.axis_index("c")
        s = lax.axis_index("s")
        wid = s * _NC + c
        z16 = jnp.zeros((16,), jnp.float32)
        o16 = jnp.ones((16,), jnp.float32)

        def zbody(i, carry):
            degloc[pl.ds(i * 16, 16)] = z16
            return carry

        lax.fori_loop(0, n_rows // 16, zbody, 0)
        pltpu.sync_copy(dst_h.at[pl.ds(wid * ept, ept)], dstb)

        def ebody(g, carry):
            d16 = dstb[pl.ds(g * 16, 16)]
            plsc.addupdate_scatter(degloc, [d16], o16)
            return carry

        lax.fori_loop(0, ept // 16, ebody, 0)
        pltpu.sync_copy(degloc, out_h.at[pl.ds(wid * n_rows, n_rows)])

    return deg_kernel(dst)


_BN = 1024


def _proj_t(xp_t, w0_t):
    kin, n = xp_t.shape
    dout = w0_t.shape[0]

    def body(w_ref, x_ref, o_ref):
        o_ref[...] = jnp.maximum(
            jnp.dot(w_ref[...], x_ref[...], preferred_element_type=jnp.float32),
            0.0)

    return pl.pallas_call(
        body,
        grid=(n // _BN,),
        in_specs=[
            pl.BlockSpec((dout, kin), lambda i: (0, 0)),
            pl.BlockSpec((kin, _BN), lambda i: (0, i)),
        ],
        out_specs=pl.BlockSpec((dout, _BN), lambda i: (0, i)),
        out_shape=jax.ShapeDtypeStruct((dout, n), jnp.float32),
    )(w0_t, xp_t)


def _norm_table(h0_t, deg_p):
    d, n = h0_t.shape

    def body(h_ref, dp_ref, t_ref, n_ref):
        deg = jnp.sum(dp_ref[...], axis=0)
        norm = lax.rsqrt(jnp.maximum(deg, 1.0))
        n_ref[...] = norm
        t_ref[...] = h_ref[...] * norm

    return pl.pallas_call(
        body,
        grid=(n // _BN,),
        in_specs=[
            pl.BlockSpec((d, _BN), lambda i: (0, i)),
            pl.BlockSpec((_NT, 1, _BN), lambda i: (0, 0, i)),
        ],
        out_specs=[
            pl.BlockSpec((d, _BN), lambda i: (0, i)),
            pl.BlockSpec((1, _BN), lambda i: (0, i)),
        ],
        out_shape=[
            jax.ShapeDtypeStruct((d, n), jnp.float32),
            jax.ShapeDtypeStruct((1, n), jnp.float32),
        ],
    )(h0_t, deg_p)


def _x1_table2(acc1_t, norm):
    d, n = acc1_t.shape

    def body(a_ref, n_ref, x_ref, t_ref):
        x1 = -(a_ref[...] * n_ref[...])
        x_ref[...] = x1
        t_ref[...] = x1 * n_ref[...]

    return pl.pallas_call(
        body,
        grid=(n // _BN,),
        in_specs=[
            pl.BlockSpec((d, _BN), lambda i: (0, i)),
            pl.BlockSpec((1, _BN), lambda i: (0, i)),
        ],
        out_specs=[
            pl.BlockSpec((d, _BN), lambda i: (0, i)),
            pl.BlockSpec((d, _BN), lambda i: (0, i)),
        ],
        out_shape=[
            jax.ShapeDtypeStruct((d, n), jnp.float32),
            jax.ShapeDtypeStruct((d, n), jnp.float32),
        ],
    )(acc1_t, norm)


def _cheb_combine(acc2_t, norm, h0_t, x1_t, cw0_t, cw1_t, cw2_t, cb_col):
    d, n = h0_t.shape

    def body(a_ref, n_ref, h0_ref, x1_ref, w0_ref, w1_ref, w2_ref, b_ref,
             o_ref):
        h0v = h0_ref[...]
        x1v = x1_ref[...]
        x2v = -2.0 * (a_ref[...] * n_ref[...]) - h0v
        acc = jnp.dot(w0_ref[...], h0v, preferred_element_type=jnp.float32)
        acc += jnp.dot(w1_ref[...], x1v, preferred_element_type=jnp.float32)
        acc += jnp.dot(w2_ref[...], x2v, preferred_element_type=jnp.float32)
        o_ref[...] = jnp.maximum(acc + b_ref[...], 0.0)

    return pl.pallas_call(
        body,
        grid=(n // _BN,),
        in_specs=[
            pl.BlockSpec((d, _BN), lambda i: (0, i)),
            pl.BlockSpec((1, _BN), lambda i: (0, i)),
            pl.BlockSpec((d, _BN), lambda i: (0, i)),
            pl.BlockSpec((d, _BN), lambda i: (0, i)),
            pl.BlockSpec((d, d), lambda i: (0, 0)),
            pl.BlockSpec((d, d), lambda i: (0, 0)),
            pl.BlockSpec((d, d), lambda i: (0, 0)),
            pl.BlockSpec((d, 1), lambda i: (0, 0)),
        ],
        out_specs=pl.BlockSpec((d, _BN), lambda i: (0, i)),
        out_shape=jax.ShapeDtypeStruct((d, n), jnp.float32),
    )(acc2_t, norm, h0_t, x1_t, cw0_t, cw1_t, cw2_t, cb_col)


def _mlp_head(hf, w1, b1, g1, bt1, w2, b2, g2, bt2, w3p, b3p, label2):
    bsz = hf.shape[0]
    inv = float((1.0 + 1e-5) ** -0.5)

    def body(h_ref, w1_ref, b1_ref, g1_ref, t1_ref, w2_ref, b2_ref, g2_ref,
             t2_ref, w3_ref, b3_ref, lb_ref, lg_ref, ls_ref):
        h1 = jnp.dot(h_ref[...], w1_ref[...], preferred_element_type=jnp.float32)
        h1 = jnp.maximum((h1 + b1_ref[...]) * inv * g1_ref[...] + t1_ref[...],
                         0.0)
        h2 = jnp.dot(h1, w2_ref[...], preferred_element_type=jnp.float32)
        h2 = jnp.maximum((h2 + b2_ref[...]) * inv * g2_ref[...] + t2_ref[...],
                         0.0)
        lg = jnp.dot(h2, w3_ref[...], preferred_element_type=jnp.float32)
        lg = lg + b3_ref[...]
        col = lax.broadcasted_iota(jnp.int32, lg.shape, 1)
        valid = col < 3
        lgm = jnp.where(valid, lg, -1e30)
        m = jnp.max(lgm, axis=1, keepdims=True)
        e = jnp.where(valid, jnp.exp(lg - m), 0.0)
        lse = jnp.log(jnp.sum(e, axis=1, keepdims=True))
        logp = lg - m - lse
        oh = jnp.logical_and(col == lb_ref[...], valid)
        picked = jnp.sum(jnp.where(oh, logp, 0.0), axis=1, keepdims=True)
        lg_ref[...] = lg
        ls_ref[...] = jnp.reshape(-jnp.mean(picked), (1, 1))

    return pl.pallas_call(
        body,
        out_shape=[
            jax.ShapeDtypeStruct((bsz, 128), jnp.float32),
            jax.ShapeDtypeStruct((1, 1), jnp.float32),
        ],
    )(hf, w1, b1, g1, bt1, w2, b2, g2, bt2, w3p, b3p, label2)


def kernel(x, edge_index, label, W0, cheb_W, cheb_b, W1, b1, g1, bt1,
           W2, b2, g2, bt2, W3, b3):
    n, in_dim = x.shape
    hid = W0.shape[1]
    bsz = label.shape[0]

    src = edge_index[0]
    dst = edge_index[1]
    assert n <= 65536
    packed = src | (dst << 16)

    xp_t = jnp.pad(x, ((0, 0), (0, 8 - in_dim))).T
    w0_t = jnp.pad(W0, ((0, 8 - in_dim), (0, 0))).T
    h0_t = _proj_t(xp_t, w0_t)

    deg_p = _deg_sc(dst, n).reshape(_NT, 1, n)
    table1_t, norm = _norm_table(h0_t, deg_p)

    acc1_t = _seg_sum_sc(table1_t.reshape(-1), packed, n, hid)
    x1_t, table2_t = _x1_table2(acc1_t.reshape(hid, n), norm)
    acc2_t = _seg_sum_sc(table2_t.reshape(-1), packed, n, hid)

    cw0_t = cheb_W[0 * hid:1 * hid].T
    cw1_t = cheb_W[1 * hid:2 * hid].T
    cw2_t = cheb_W[2 * hid:3 * hid].T
    h_t = _cheb_combine(acc2_t.reshape(hid, n), norm, h0_t, x1_t,
                        cw0_t, cw1_t, cw2_t, cheb_b[:, None])

    hf = h_t.T.reshape(bsz, -1)
    w3p = jnp.pad(W3, ((0, 0), (0, 128 - W3.shape[1])))
    b3p = jnp.pad(b3, (0, 128 - b3.shape[0]))[None, :]
    logits_pad, loss = _mlp_head(
        hf, W1, b1[None, :], g1[None, :], bt1[None, :],
        W2, b2[None, :], g2[None, :], bt2[None, :],
        w3p, b3p, label[:, None].astype(jnp.int32))

    return (logits_pad[:, :W3.shape[1]], loss[0, 0])


# speedup vs baseline: 2.9256x; 2.9256x over previous
import functools

import jax
import jax.numpy as jnp
from jax import lax
from jax.experimental import pallas as pl
from jax.experimental.pallas import tpu as pltpu
from jax.experimental.pallas import tpu_sc as plsc

_NC = 2
_NS = 16
_NT = _NC * _NS
_ECH = 7936


def _seg_sum_sc(table_t, packed, n_rows, d):
    e = packed.shape[0]
    unroll = 8
    assert e % _ECH == 0 and _ECH % (16 * unroll) == 0
    n_chunks = e // _ECH
    n_pass = d // _NT
    assert d % _NT == 0

    mesh = plsc.VectorSubcoreMesh(core_axis_name="c", subcore_axis_name="s")

    @functools.partial(
        pl.kernel,
        mesh=mesh,
        out_type=jax.ShapeDtypeStruct((d * n_rows,), jnp.float32),
        scratch_types=[
            pltpu.VMEM((n_rows,), jnp.float32),
            pltpu.VMEM((n_rows,), jnp.float32),
            pltpu.VMEM((2, _ECH), jnp.int32),
            pltpu.SemaphoreType.DMA,
        ],
        compiler_params=pltpu.CompilerParams(needs_layout_passes=False),
    )
    def seg_kernel(table_h, pk_h, out_h, tloc, aloc, pkb, sem):
        c = lax.axis_index("c")
        s = lax.axis_index("s")
        wid = s * _NC + c
        z16 = jnp.zeros((16,), jnp.float32)

        for p in range(n_pass):
            col = wid + _NT * p
            pltpu.sync_copy(table_h.at[pl.ds(col * n_rows, n_rows)], tloc)

            def zbody(i, carry):
                aloc[pl.ds(i * 16, 16)] = z16
                return carry

            lax.fori_loop(0, n_rows // 16, zbody, 0)

            pltpu.async_copy(pk_h.at[pl.ds(0, _ECH)], pkb.at[0], sem)

            def chunk_body(i, carry):
                par = i % 2
                nxt = (i + 1) % 2
                pltpu.make_async_copy(
                    pk_h.at[pl.ds(i * _ECH, _ECH)], pkb.at[par], sem).wait()

                @pl.when(i + 1 < n_chunks)
                def _prefetch():
                    pltpu.async_copy(
                        pk_h.at[pl.ds((i + 1) * _ECH, _ECH)], pkb.at[nxt],
                        sem)

                @plsc.parallel_loop(0, _ECH // 16, unroll=unroll)
                def _edges(g):
                    w = pkb[par, pl.ds(g * 16, 16)]
                    s16 = jnp.bitwise_and(w, 0xFFFF)
                    d16 = lax.shift_right_logical(w, 16)
                    v = plsc.load_gather(tloc, [s16])
                    plsc.addupdate_scatter(aloc, [d16], v)

                return carry

            lax.fori_loop(0, n_chunks, chunk_body, 0)
            pltpu.sync_copy(aloc, out_h.at[pl.ds(col * n_rows, n_rows)])

    return seg_kernel(table_t, packed)


def _deg_sc(dst, n_rows):
    e = dst.shape[0]
    ept = e // _NT
    assert e % _NT == 0 and ept % 16 == 0

    mesh = plsc.VectorSubcoreMesh(core_axis_name="c", subcore_axis_name="s")

    @functools.partial(
        pl.kernel,
        mesh=mesh,
        out_type=jax.ShapeDtypeStruct((_NT * n_rows,), jnp.float32),
        scratch_types=[
            pltpu.VMEM((n_rows,), jnp.float32),
            pltpu.VMEM((ept,), jnp.int32),
            pltpu.SemaphoreType.DMA,
        ],
        compiler_params=pltpu.CompilerParams(needs_layout_passes=False),
    )
    def deg_kernel(dst_h, out_h, degloc, dstb, sem):
        c = lax.axis_index("c")
        s = lax.axis_index("s")
        wid = s * _NC + c
        z16 = jnp.zeros((16,), jnp.float32)
        o16 = jnp.ones((16,), jnp.float32)

        def zbody(i, carry):
            degloc[pl.ds(i * 16, 16)] = z16
            return carry

        lax.fori_loop(0, n_rows // 16, zbody, 0)
        pltpu.sync_copy(dst_h.at[pl.ds(wid * ept, ept)], dstb)

        def ebody(g, carry):
            d16 = dstb[pl.ds(g * 16, 16)]
            plsc.addupdate_scatter(degloc, [d16], o16)
            return carry

        lax.fori_loop(0, ept // 16, ebody, 0)
        pltpu.sync_copy(degloc, out_h.at[pl.ds(wid * n_rows, n_rows)])

    return deg_kernel(dst)


_BN = 1024


def _proj_t(xp_t, w0_t):
    kin, n = xp_t.shape
    dout = w0_t.shape[0]

    def body(w_ref, x_ref, o_ref):
        o_ref[...] = jnp.maximum(
            jnp.dot(w_ref[...], x_ref[...], preferred_element_type=jnp.float32),
            0.0)

    return pl.pallas_call(
        body,
        grid=(n // _BN,),
        in_specs=[
            pl.BlockSpec((dout, kin), lambda i: (0, 0)),
            pl.BlockSpec((kin, _BN), lambda i: (0, i)),
        ],
        out_specs=pl.BlockSpec((dout, _BN), lambda i: (0, i)),
        out_shape=jax.ShapeDtypeStruct((dout, n), jnp.float32),
    )(w0_t, xp_t)


def _norm_table(h0_t, deg_p):
    d, n = h0_t.shape

    def body(h_ref, dp_ref, t_ref, n_ref):
        deg = jnp.sum(dp_ref[...], axis=0)
        norm = lax.rsqrt(jnp.maximum(deg, 1.0))
        n_ref[...] = norm
        t_ref[...] = h_ref[...] * norm

    return pl.pallas_call(
        body,
        grid=(n // _BN,),
        in_specs=[
            pl.BlockSpec((d, _BN), lambda i: (0, i)),
            pl.BlockSpec((_NT, 1, _BN), lambda i: (0, 0, i)),
        ],
        out_specs=[
            pl.BlockSpec((d, _BN), lambda i: (0, i)),
            pl.BlockSpec((1, _BN), lambda i: (0, i)),
        ],
        out_shape=[
            jax.ShapeDtypeStruct((d, n), jnp.float32),
            jax.ShapeDtypeStruct((1, n), jnp.float32),
        ],
    )(h0_t, deg_p)


def _x1_table2(acc1_t, norm):
    d, n = acc1_t.shape

    def body(a_ref, n_ref, x_ref, t_ref):
        x1 = -(a_ref[...] * n_ref[...])
        x_ref[...] = x1
        t_ref[...] = x1 * n_ref[...]

    return pl.pallas_call(
        body,
        grid=(n // _BN,),
        in_specs=[
            pl.BlockSpec((d, _BN), lambda i: (0, i)),
            pl.BlockSpec((1, _BN), lambda i: (0, i)),
        ],
        out_specs=[
            pl.BlockSpec((d, _BN), lambda i: (0, i)),
            pl.BlockSpec((d, _BN), lambda i: (0, i)),
        ],
        out_shape=[
            jax.ShapeDtypeStruct((d, n), jnp.float32),
            jax.ShapeDtypeStruct((d, n), jnp.float32),
        ],
    )(acc1_t, norm)


def _cheb_combine(acc2_t, norm, h0_t, x1_t, cw0_t, cw1_t, cw2_t, cb_col):
    d, n = h0_t.shape

    def body(a_ref, n_ref, h0_ref, x1_ref, w0_ref, w1_ref, w2_ref, b_ref,
             o_ref):
        h0v = h0_ref[...]
        x1v = x1_ref[...]
        x2v = -2.0 * (a_ref[...] * n_ref[...]) - h0v
        acc = jnp.dot(w0_ref[...], h0v, preferred_element_type=jnp.float32)
        acc += jnp.dot(w1_ref[...], x1v, preferred_element_type=jnp.float32)
        acc += jnp.dot(w2_ref[...], x2v, preferred_element_type=jnp.float32)
        o_ref[...] = jnp.maximum(acc + b_ref[...], 0.0)

    return pl.pallas_call(
        body,
        grid=(n // _BN,),
        in_specs=[
            pl.BlockSpec((d, _BN), lambda i: (0, i)),
            pl.BlockSpec((1, _BN), lambda i: (0, i)),
            pl.BlockSpec((d, _BN), lambda i: (0, i)),
            pl.BlockSpec((d, _BN), lambda i: (0, i)),
            pl.BlockSpec((d, d), lambda i: (0, 0)),
            pl.BlockSpec((d, d), lambda i: (0, 0)),
            pl.BlockSpec((d, d), lambda i: (0, 0)),
            pl.BlockSpec((d, 1), lambda i: (0, 0)),
        ],
        out_specs=pl.BlockSpec((d, _BN), lambda i: (0, i)),
        out_shape=jax.ShapeDtypeStruct((d, n), jnp.float32),
    )(acc2_t, norm, h0_t, x1_t, cw0_t, cw1_t, cw2_t, cb_col)


def _mlp_head(hf, w1, b1, g1, bt1, w2, b2, g2, bt2, w3p, b3p, label2):
    bsz = hf.shape[0]
    inv = float((1.0 + 1e-5) ** -0.5)

    def body(h_ref, w1_ref, b1_ref, g1_ref, t1_ref, w2_ref, b2_ref, g2_ref,
             t2_ref, w3_ref, b3_ref, lb_ref, lg_ref, ls_ref):
        h1 = jnp.dot(h_ref[...], w1_ref[...], preferred_element_type=jnp.float32)
        h1 = jnp.maximum((h1 + b1_ref[...]) * inv * g1_ref[...] + t1_ref[...],
                         0.0)
        h2 = jnp.dot(h1, w2_ref[...], preferred_element_type=jnp.float32)
        h2 = jnp.maximum((h2 + b2_ref[...]) * inv * g2_ref[...] + t2_ref[...],
                         0.0)
        lg = jnp.dot(h2, w3_ref[...], preferred_element_type=jnp.float32)
        lg = lg + b3_ref[...]
        col = lax.broadcasted_iota(jnp.int32, lg.shape, 1)
        valid = col < 3
        lgm = jnp.where(valid, lg, -1e30)
        m = jnp.max(lgm, axis=1, keepdims=True)
        e = jnp.where(valid, jnp.exp(lg - m), 0.0)
        lse = jnp.log(jnp.sum(e, axis=1, keepdims=True))
        logp = lg - m - lse
        oh = jnp.logical_and(col == lb_ref[...], valid)
        picked = jnp.sum(jnp.where(oh, logp, 0.0), axis=1, keepdims=True)
        lg_ref[...] = lg
        ls_ref[...] = jnp.reshape(-jnp.mean(picked), (1, 1))

    return pl.pallas_call(
        body,
        out_shape=[
            jax.ShapeDtypeStruct((bsz, 128), jnp.float32),
            jax.ShapeDtypeStruct((1, 1), jnp.float32),
        ],
    )(hf, w1, b1, g1, bt1, w2, b2, g2, bt2, w3p, b3p, label2)


def kernel(x, edge_index, label, W0, cheb_W, cheb_b, W1, b1, g1, bt1,
           W2, b2, g2, bt2, W3, b3):
    n, in_dim = x.shape
    hid = W0.shape[1]
    bsz = label.shape[0]

    src = edge_index[0]
    dst = edge_index[1]
    assert n <= 65536
    packed = src | (dst << 16)

    xp_t = jnp.pad(x, ((0, 0), (0, 8 - in_dim))).T
    w0_t = jnp.pad(W0, ((0, 8 - in_dim), (0, 0))).T
    h0_t = _proj_t(xp_t, w0_t)

    deg_p = _deg_sc(dst, n).reshape(_NT, 1, n)
    table1_t, norm = _norm_table(h0_t, deg_p)

    acc1_t = _seg_sum_sc(table1_t.reshape(-1), packed, n, hid)
    x1_t, table2_t = _x1_table2(acc1_t.reshape(hid, n), norm)
    acc2_t = _seg_sum_sc(table2_t.reshape(-1), packed, n, hid)

    cw0_t = cheb_W[0 * hid:1 * hid].T
    cw1_t = cheb_W[1 * hid:2 * hid].T
    cw2_t = cheb_W[2 * hid:3 * hid].T
    h_t = _cheb_combine(acc2_t.reshape(hid, n), norm, h0_t, x1_t,
                        cw0_t, cw1_t, cw2_t, cheb_b[:, None])

    hf = h_t.T.reshape(bsz, -1)
    w3p = jnp.pad(W3, ((0, 0), (0, 128 - W3.shape[1])))
    b3p = jnp.pad(b3, (0, 128 - b3.shape[0]))[None, :]
    logits_pad, loss = _mlp_head(
        hf, W1, b1[None, :], g1[None, :], bt1[None, :],
        W2, b2[None, :], g2[None, :], bt2[None, :],
        w3p, b3p, label[:, None].astype(jnp.int32))

    return (logits_pad[:, :W3.shape[1]], loss[0, 0])


# speedup vs baseline: 2.9532x; 1.0094x over previous
import functools

import jax
import jax.numpy as jnp
from jax import lax
from jax.experimental import pallas as pl
from jax.experimental.pallas import tpu as pltpu
from jax.experimental.pallas import tpu_sc as plsc

_NC = 2
_NS = 16
_NT = _NC * _NS
_ECH = 7936


def _seg_sum_sc(table_t, packed, n_rows, d):
    e = packed.shape[0]
    unroll = 16
    assert e % _ECH == 0 and _ECH % (16 * unroll) == 0
    n_chunks = e // _ECH
    n_pass = d // _NT
    assert d % _NT == 0

    mesh = plsc.VectorSubcoreMesh(core_axis_name="c", subcore_axis_name="s")

    @functools.partial(
        pl.kernel,
        mesh=mesh,
        out_type=jax.ShapeDtypeStruct((d * n_rows,), jnp.float32),
        scratch_types=[
            pltpu.VMEM((n_rows,), jnp.float32),
            pltpu.VMEM((n_rows,), jnp.float32),
            pltpu.VMEM((2, _ECH), jnp.int32),
            pltpu.SemaphoreType.DMA,
        ],
        compiler_params=pltpu.CompilerParams(needs_layout_passes=False),
    )
    def seg_kernel(table_h, pk_h, out_h, tloc, aloc, pkb, sem):
        c = lax.axis_index("c")
        s = lax.axis_index("s")
        wid = s * _NC + c
        z16 = jnp.zeros((16,), jnp.float32)

        for p in range(n_pass):
            col = wid + _NT * p
            pltpu.sync_copy(table_h.at[pl.ds(col * n_rows, n_rows)], tloc)

            def zbody(i, carry):
                aloc[pl.ds(i * 16, 16)] = z16
                return carry

            lax.fori_loop(0, n_rows // 16, zbody, 0)

            pltpu.async_copy(pk_h.at[pl.ds(0, _ECH)], pkb.at[0], sem)

            def chunk_body(i, carry):
                par = i % 2
                nxt = (i + 1) % 2
                pltpu.make_async_copy(
                    pk_h.at[pl.ds(i * _ECH, _ECH)], pkb.at[par], sem).wait()

                @pl.when(i + 1 < n_chunks)
                def _prefetch():
                    pltpu.async_copy(
                        pk_h.at[pl.ds((i + 1) * _ECH, _ECH)], pkb.at[nxt],
                        sem)

                @plsc.parallel_loop(0, _ECH // 16, unroll=unroll)
                def _edges(g):
                    w = pkb[par, pl.ds(g * 16, 16)]
                    s16 = jnp.bitwise_and(w, 0xFFFF)
                    d16 = lax.shift_right_logical(w, 16)
                    v = plsc.load_gather(tloc, [s16])
                    plsc.addupdate_scatter(aloc, [d16], v)

                return carry

            lax.fori_loop(0, n_chunks, chunk_body, 0)
            pltpu.sync_copy(aloc, out_h.at[pl.ds(col * n_rows, n_rows)])

    return seg_kernel(table_t, packed)


def _deg_sc(dst, n_rows):
    e = dst.shape[0]
    ept = e // _NT
    assert e % _NT == 0 and ept % 16 == 0

    mesh = plsc.VectorSubcoreMesh(core_axis_name="c", subcore_axis_name="s")

    @functools.partial(
        pl.kernel,
        mesh=mesh,
        out_type=jax.ShapeDtypeStruct((_NT * n_rows,), jnp.float32),
        scratch_types=[
            pltpu.VMEM((n_rows,), jnp.float32),
            pltpu.VMEM((ept,), jnp.int32),
            pltpu.SemaphoreType.DMA,
        ],
        compiler_params=pltpu.CompilerParams(needs_layout_passes=False),
    )
    def deg_kernel(dst_h, out_h, degloc, dstb, sem):
        c = lax.axis_index("c")
        s = lax.axis_index("s")
        wid = s * _NC + c
        z16 = jnp.zeros((16,), jnp.float32)
        o16 = jnp.ones((16,), jnp.float32)

        def zbody(i, carry):
            degloc[pl.ds(i * 16, 16)] = z16
            return carry

        lax.fori_loop(0, n_rows // 16, zbody, 0)
        pltpu.sync_copy(dst_h.at[pl.ds(wid * ept, ept)], dstb)

        def ebody(g, carry):
            d16 = dstb[pl.ds(g * 16, 16)]
            plsc.addupdate_scatter(degloc, [d16], o16)
            return carry

        lax.fori_loop(0, ept // 16, ebody, 0)
        pltpu.sync_copy(degloc, out_h.at[pl.ds(wid * n_rows, n_rows)])

    return deg_kernel(dst)


_BN = 1024


def _proj_t(xp_t, w0_t):
    kin, n = xp_t.shape
    dout = w0_t.shape[0]

    def body(w_ref, x_ref, o_ref):
        o_ref[...] = jnp.maximum(
            jnp.dot(w_ref[...], x_ref[...], preferred_element_type=jnp.float32),
            0.0)

    return pl.pallas_call(
        body,
        grid=(n // _BN,),
        in_specs=[
            pl.BlockSpec((dout, kin), lambda i: (0, 0)),
            pl.BlockSpec((kin, _BN), lambda i: (0, i)),
        ],
        out_specs=pl.BlockSpec((dout, _BN), lambda i: (0, i)),
        out_shape=jax.ShapeDtypeStruct((dout, n), jnp.float32),
    )(w0_t, xp_t)


def _norm_table(h0_t, deg_p):
    d, n = h0_t.shape

    def body(h_ref, dp_ref, t_ref, n_ref):
        deg = jnp.sum(dp_ref[...], axis=0)
        norm = lax.rsqrt(jnp.maximum(deg, 1.0))
        n_ref[...] = norm
        t_ref[...] = h_ref[...] * norm

    return pl.pallas_call(
        body,
        grid=(n // _BN,),
        in_specs=[
            pl.BlockSpec((d, _BN), lambda i: (0, i)),
            pl.BlockSpec((_NT, 1, _BN), lambda i: (0, 0, i)),
        ],
        out_specs=[
            pl.BlockSpec((d, _BN), lambda i: (0, i)),
            pl.BlockSpec((1, _BN), lambda i: (0, i)),
        ],
        out_shape=[
            jax.ShapeDtypeStruct((d, n), jnp.float32),
            jax.ShapeDtypeStruct((1, n), jnp.float32),
        ],
    )(h0_t, deg_p)


def _x1_table2(acc1_t, norm):
    d, n = acc1_t.shape

    def body(a_ref, n_ref, x_ref, t_ref):
        x1 = -(a_ref[...] * n_ref[...])
        x_ref[...] = x1
        t_ref[...] = x1 * n_ref[...]

    return pl.pallas_call(
        body,
        grid=(n // _BN,),
        in_specs=[
            pl.BlockSpec((d, _BN), lambda i: (0, i)),
            pl.BlockSpec((1, _BN), lambda i: (0, i)),
        ],
        out_specs=[
            pl.BlockSpec((d, _BN), lambda i: (0, i)),
            pl.BlockSpec((d, _BN), lambda i: (0, i)),
        ],
        out_shape=[
            jax.ShapeDtypeStruct((d, n), jnp.float32),
            jax.ShapeDtypeStruct((d, n), jnp.float32),
        ],
    )(acc1_t, norm)


def _cheb_combine(acc2_t, norm, h0_t, x1_t, cw0_t, cw1_t, cw2_t, cb_col):
    d, n = h0_t.shape

    def body(a_ref, n_ref, h0_ref, x1_ref, w0_ref, w1_ref, w2_ref, b_ref,
             o_ref):
        h0v = h0_ref[...]
        x1v = x1_ref[...]
        x2v = -2.0 * (a_ref[...] * n_ref[...]) - h0v
        acc = jnp.dot(w0_ref[...], h0v, preferred_element_type=jnp.float32)
        acc += jnp.dot(w1_ref[...], x1v, preferred_element_type=jnp.float32)
        acc += jnp.dot(w2_ref[...], x2v, preferred_element_type=jnp.float32)
        o_ref[...] = jnp.maximum(acc + b_ref[...], 0.0)

    return pl.pallas_call(
        body,
        grid=(n // _BN,),
        in_specs=[
            pl.BlockSpec((d, _BN), lambda i: (0, i)),
            pl.BlockSpec((1, _BN), lambda i: (0, i)),
            pl.BlockSpec((d, _BN), lambda i: (0, i)),
            pl.BlockSpec((d, _BN), lambda i: (0, i)),
            pl.BlockSpec((d, d), lambda i: (0, 0)),
            pl.BlockSpec((d, d), lambda i: (0, 0)),
            pl.BlockSpec((d, d), lambda i: (0, 0)),
            pl.BlockSpec((d, 1), lambda i: (0, 0)),
        ],
        out_specs=pl.BlockSpec((d, _BN), lambda i: (0, i)),
        out_shape=jax.ShapeDtypeStruct((d, n), jnp.float32),
    )(acc2_t, norm, h0_t, x1_t, cw0_t, cw1_t, cw2_t, cb_col)


def _mlp_head(hf, w1, b1, g1, bt1, w2, b2, g2, bt2, w3p, b3p, label2):
    bsz = hf.shape[0]
    inv = float((1.0 + 1e-5) ** -0.5)

    def body(h_ref, w1_ref, b1_ref, g1_ref, t1_ref, w2_ref, b2_ref, g2_ref,
             t2_ref, w3_ref, b3_ref, lb_ref, lg_ref, ls_ref):
        h1 = jnp.dot(h_ref[...], w1_ref[...], preferred_element_type=jnp.float32)
        h1 = jnp.maximum((h1 + b1_ref[...]) * inv * g1_ref[...] + t1_ref[...],
                         0.0)
        h2 = jnp.dot(h1, w2_ref[...], preferred_element_type=jnp.float32)
        h2 = jnp.maximum((h2 + b2_ref[...]) * inv * g2_ref[...] + t2_ref[...],
                         0.0)
        lg = jnp.dot(h2, w3_ref[...], preferred_element_type=jnp.float32)
        lg = lg + b3_ref[...]
        col = lax.broadcasted_iota(jnp.int32, lg.shape, 1)
        valid = col < 3
        lgm = jnp.where(valid, lg, -1e30)
        m = jnp.max(lgm, axis=1, keepdims=True)
        e = jnp.where(valid, jnp.exp(lg - m), 0.0)
        lse = jnp.log(jnp.sum(e, axis=1, keepdims=True))
        logp = lg - m - lse
        oh = jnp.logical_and(col == lb_ref[...], valid)
        picked = jnp.sum(jnp.where(oh, logp, 0.0), axis=1, keepdims=True)
        lg_ref[...] = lg
        ls_ref[...] = jnp.reshape(-jnp.mean(picked), (1, 1))

    return pl.pallas_call(
        body,
        out_shape=[
            jax.ShapeDtypeStruct((bsz, 128), jnp.float32),
            jax.ShapeDtypeStruct((1, 1), jnp.float32),
        ],
    )(hf, w1, b1, g1, bt1, w2, b2, g2, bt2, w3p, b3p, label2)


def kernel(x, edge_index, label, W0, cheb_W, cheb_b, W1, b1, g1, bt1,
           W2, b2, g2, bt2, W3, b3):
    n, in_dim = x.shape
    hid = W0.shape[1]
    bsz = label.shape[0]

    src = edge_index[0]
    dst = edge_index[1]
    assert n <= 65536
    packed = src | (dst << 16)

    xp_t = jnp.pad(x, ((0, 0), (0, 8 - in_dim))).T
    w0_t = jnp.pad(W0, ((0, 8 - in_dim), (0, 0))).T
    h0_t = _proj_t(xp_t, w0_t)

    deg_p = _deg_sc(dst, n).reshape(_NT, 1, n)
    table1_t, norm = _norm_table(h0_t, deg_p)

    acc1_t = _seg_sum_sc(table1_t.reshape(-1), packed, n, hid)
    x1_t, table2_t = _x1_table2(acc1_t.reshape(hid, n), norm)
    acc2_t = _seg_sum_sc(table2_t.reshape(-1), packed, n, hid)

    cw0_t = cheb_W[0 * hid:1 * hid].T
    cw1_t = cheb_W[1 * hid:2 * hid].T
    cw2_t = cheb_W[2 * hid:3 * hid].T
    h_t = _cheb_combine(acc2_t.reshape(hid, n), norm, h0_t, x1_t,
                        cw0_t, cw1_t, cw2_t, cheb_b[:, None])

    hf = h_t.T.reshape(bsz, -1)
    w3p = jnp.pad(W3, ((0, 0), (0, 128 - W3.shape[1])))
    b3p = jnp.pad(b3, (0, 128 - b3.shape[0]))[None, :]
    logits_pad, loss = _mlp_head(
        hf, W1, b1[None, :], g1[None, :], bt1[None, :],
        W2, b2[None, :], g2[None, :], bt2[None, :],
        w3p, b3p, label[:, None].astype(jnp.int32))

    return (logits_pad[:, :W3.shape[1]], loss[0, 0])
